# baseline (speedup 1.0000x reference)
import functools

import jax
import jax.numpy as jnp
from jax import lax
from jax.experimental import pallas as pl
from jax.experimental.pallas import tpu as pltpu
from jax.experimental.pallas import tpu_sc as plsc

B = 4096
F = 26
V = 1000000
D = 16
NUM = 13
H1 = 256
H2 = 128
EPS = 1e-5

NC = 2
NS = 16
NW = NC * NS
NP = F * D
PPW = NP // NW
NST = B // 128

_mesh = plsc.VectorSubcoreMesh(core_axis_name="c", subcore_axis_name="s")


@functools.partial(
    pl.kernel,
    mesh=_mesh,
    out_type=jax.ShapeDtypeStruct((NP * B,), jnp.float32),
    scratch_types=[
        pltpu.VMEM((2, NST, 128), jnp.int32),
        pltpu.VMEM((2, B), jnp.float32),
        pltpu.SemaphoreType.DMA,
        pltpu.SemaphoreType.DMA,
    ],
    compiler_params=pltpu.CompilerParams(use_tc_tiling_on_sc=False),
)
def _sc_gather(table_hbm, xcat_hbm, out_hbm, idx_v, ring_v, sem, osem):
    wid = lax.axis_index("s") * NC + lax.axis_index("c")
    p0 = wid * PPW
    f0 = p0 >> 4
    f1 = jnp.minimum(f0 + 1, F - 1)
    pltpu.sync_copy(xcat_hbm.at[f0], idx_v.at[0])
    pltpu.sync_copy(xcat_hbm.at[f1], idx_v.at[1])

    def fire(k):
        p = p0 + k
        f = p >> 4
        d = p & 15
        fsel = f - f0
        slot = k % 2
        for c in range(NST):
            pltpu.make_async_copy(
                table_hbm.at[f, d].at[idx_v.at[fsel, c]],
                ring_v.at[slot, pl.ds(c * 128, 128)],
                sem,
            ).start()

    def drain(k):
        slot = k % 2
        for c in range(NST):
            pltpu.make_async_copy(
                table_hbm.at[0, 0].at[idx_v.at[0, c]],
                ring_v.at[slot, pl.ds(c * 128, 128)],
                sem,
            ).wait()
        pltpu.make_async_copy(
            ring_v.at[slot], out_hbm.at[pl.ds((p0 + k) * B, B)], osem,
        ).start()

    fire(0)
    for k in range(1, PPW):
        if k >= 2:
            pltpu.make_async_copy(
                ring_v.at[0], out_hbm.at[pl.ds(0, B)], osem).wait()
        fire(k)
        drain(k - 1)
    drain(PPW - 1)
    pltpu.make_async_copy(
        ring_v.at[0], out_hbm.at[pl.ds(0, B)], osem).wait()
    pltpu.make_async_copy(
        ring_v.at[0], out_hbm.at[pl.ds(0, B)], osem).wait()


def _mlp_body(xg_ref, xn_ref,
              bg_ref, bb_ref, bm_ref, bv_ref,
              w0c_ref, w0n_ref, b0_ref, g0_ref, be0_ref, m0_ref, v0_ref,
              w1_ref, b1_ref, g1_ref, be1_ref, m1_ref, v1_ref,
              w2_ref, b2_ref, out_ref):
    xn = xn_ref[...]
    xnb = (xn - bm_ref[...]) * lax.rsqrt(bv_ref[...] + EPS) * bg_ref[...] + bb_ref[...]
    h = lax.dot_general(xg_ref[...], w0c_ref[...], (((0,), (0,)), ((), ())),
                        preferred_element_type=jnp.float32)
    h = h + jnp.dot(xnb, w0n_ref[...], preferred_element_type=jnp.float32)
    h = jnp.maximum(h + b0_ref[...], 0.0)
    h = (h - m0_ref[...]) * lax.rsqrt(v0_ref[...] + EPS) * g0_ref[...] + be0_ref[...]
    h = jnp.dot(h, w1_ref[...], preferred_element_type=jnp.float32)
    h = jnp.maximum(h + b1_ref[...], 0.0)
    h = (h - m1_ref[...]) * lax.rsqrt(v1_ref[...] + EPS) * g1_ref[...] + be1_ref[...]
    out_ref[...] = jnp.dot(h, w2_ref[...], preferred_element_type=jnp.float32) + b2_ref[...]


def _tc_mlp(xg, xn, bg, bb, bm, bv, w0c, w0n, b0, g0, be0, m0, v0,
            w1, b1, g1, be1, m1, v1, w2, b2):
    TB = 512
    grid = (B // TB,)
    col = lambda i: (0, i)
    row = lambda i: (i, 0)
    rep = lambda i: (0, 0)
    full = lambda a: pl.BlockSpec(a.shape, rep)
    return pl.pallas_call(
        _mlp_body,
        grid=grid,
        in_specs=[
            pl.BlockSpec((NP, TB), col),
            pl.BlockSpec((TB, NUM), row),
            full(bg), full(bb), full(bm), full(bv),
            full(w0c), full(w0n), full(b0), full(g0), full(be0), full(m0), full(v0),
            full(w1), full(b1), full(g1), full(be1), full(m1), full(v1),
            full(w2), full(b2),
        ],
        out_specs=pl.BlockSpec((TB, 1), row),
        out_shape=jax.ShapeDtypeStruct((B, 1), jnp.float32),
    )(xg, xn, bg, bb, bm, bv, w0c, w0n, b0, g0, be0, m0, v0,
      w1, b1, g1, be1, m1, v1, w2, b2)


def kernel(x_categorical, x_numerical, emb_tables, bn_num_gamma, bn_num_beta,
           bn_num_mean, bn_num_var, w0, b0, g0, be0, m0, v0,
           w1, b1, g1, be1, m1, v1, w2, b2):
    xcat = x_categorical.astype(jnp.int32).T.reshape(F, B // 128, 128)
    table3 = jnp.swapaxes(emb_tables, 1, 2)
    gathered = _sc_gather(table3, xcat)
    xg = gathered.reshape(NP, B)

    r2 = lambda a: a.reshape(1, -1)
    return _tc_mlp(
        xg, x_numerical,
        r2(bn_num_gamma), r2(bn_num_beta), r2(bn_num_mean), r2(bn_num_var),
        w0[:, :NP].T, w0[:, NP:].T, r2(b0), r2(g0), r2(be0), r2(m0), r2(v0),
        w1.T, r2(b1), r2(g1), r2(be1), r2(m1), r2(v1),
        w2.T, r2(b2),
    )


# speedup vs baseline: 4.1287x; 4.1287x over previous
import functools

import jax
import jax.numpy as jnp
from jax import lax
from jax.experimental import pallas as pl
from jax.experimental.pallas import tpu as pltpu
from jax.experimental.pallas import tpu_sc as plsc

B = 4096
F = 26
V = 1000000
D = 16
NUM = 13
H1 = 256
H2 = 128
EPS = 1e-5

NC = 2
NS = 16
NW = NC * NS
R = B * F
RPW = R // NW
CH = 128
NCH = RPW // CH
SROWS = F * V * D // 128

_mesh = plsc.VectorSubcoreMesh(core_axis_name="c", subcore_axis_name="s")


BV = 7936
NJ = 126
NSTEPS = F * NJ
VTAIL = V - NJ * BV


def _tr_body(x_hbm, o_ref, buf, sem):
    f = pl.program_id(0)
    j = pl.program_id(1)
    k = f * NJ + j

    @pl.when(k == 0)
    def _():
        pltpu.make_async_copy(
            x_hbm.at[pl.ds(0, D), pl.ds(0, BV)], buf.at[0], sem).start()

    @pl.when(k + 1 < NSTEPS)
    def _():
        k1 = k + 1
        nf = k1 // NJ
        nj = k1 % NJ
        pltpu.make_async_copy(
            x_hbm.at[pl.ds(pl.multiple_of(nf * D, 8), D),
                     pl.ds(pl.multiple_of(nj * BV, 128), BV)],
            buf.at[lax.rem(k1, 2)], sem).start()

    pltpu.make_async_copy(
        x_hbm.at[pl.ds(0, D), pl.ds(0, BV)], buf.at[lax.rem(k, 2)], sem).wait()
    y = buf[lax.rem(k, 2)].T.reshape(BV // 8, 8, D)
    o = jnp.concatenate([y[:, m, :] for m in range(8)], axis=-1)
    o_ref[...] = o.reshape(1, BV // 8, 128)


def _tc_transpose(table2):
    grid = (F, NJ)
    return pl.pallas_call(
        _tr_body,
        grid=grid,
        in_specs=[pl.BlockSpec(memory_space=pl.ANY)],
        out_specs=pl.BlockSpec((1, BV // 8, 128), lambda f, j: (f, j, 0)),
        out_shape=jax.ShapeDtypeStruct((F, V * D // 128, 128), jnp.float32),
        scratch_shapes=[
            pltpu.VMEM((2, D, BV), jnp.float32),
            pltpu.SemaphoreType.DMA,
        ],
    )(table2)


@functools.partial(
    pl.kernel,
    mesh=_mesh,
    out_type=jax.ShapeDtypeStruct((R * D,), jnp.float32),
    scratch_types=[
        pltpu.VMEM((RPW,), jnp.int32),
        pltpu.VMEM((2, CH), jnp.int32),
        pltpu.VMEM((2, CH, 128), jnp.float32),
        pltpu.VMEM((RPW * D,), jnp.float32),
        pltpu.SemaphoreType.DMA,
    ],
    compiler_params=pltpu.CompilerParams(
        use_tc_tiling_on_sc=False, needs_layout_passes=False),
)
def _sc_gather(table_hbm, e_hbm, out_hbm, e_v, q_v, ring_v, rows_v, sem):
    wid = lax.axis_index("s") * NC + lax.axis_index("c")
    base = wid * RPW
    pltpu.sync_copy(e_hbm.at[pl.ds(base, RPW)], e_v)
    iota = lax.iota(jnp.int32, 16)

    def fire(g):
        slot = lax.rem(g, 2)
        for j in range(CH // 16):
            q16 = e_v[pl.ds(g * CH + j * 16, 16)] >> 3
            q_v[slot, pl.ds(j * 16, 16)] = q16
        pltpu.make_async_copy(
            table_hbm.at[q_v.at[slot]], ring_v.at[slot], sem).start()

    def drain(g):
        slot = lax.rem(g, 2)
        pltpu.make_async_copy(
            table_hbm.at[q_v.at[slot]], ring_v.at[slot], sem).wait()
        for j in range(CH // 16):
            m16 = (e_v[pl.ds(g * CH + j * 16, 16)] & 7) * 16
            for t in range(16):
                jj = j * 16 + t
                vals = plsc.load_gather(
                    ring_v, [jnp.broadcast_to(slot, (16,)),
                             jnp.broadcast_to(jj, (16,)),
                             m16[t] + iota])
                plsc.store_scatter(rows_v, [(g * CH + jj) * D + iota], vals)

    fire(0)

    def body(g, _):
        fire(g)
        drain(g - 1)
        return 0

    lax.fori_loop(1, NCH, body, 0)
    drain(NCH - 1)
    pltpu.sync_copy(rows_v, out_hbm.at[pl.ds(base * D, RPW * D)])


def _mlp_body(xc_ref, xn_ref,
              bg_ref, bb_ref, bm_ref, bv_ref,
              w0c_ref, w0n_ref, b0_ref, g0_ref, be0_ref, m0_ref, v0_ref,
              w1_ref, b1_ref, g1_ref, be1_ref, m1_ref, v1_ref,
              w2_ref, b2_ref, out_ref):
    xn = xn_ref[...]
    xnb = (xn - bm_ref[...]) * lax.rsqrt(bv_ref[...] + EPS) * bg_ref[...] + bb_ref[...]
    h = jnp.dot(xc_ref[...], w0c_ref[...], preferred_element_type=jnp.float32)
    h = h + jnp.dot(xnb, w0n_ref[...], preferred_element_type=jnp.float32)
    h = jnp.maximum(h + b0_ref[...], 0.0)
    h = (h - m0_ref[...]) * lax.rsqrt(v0_ref[...] + EPS) * g0_ref[...] + be0_ref[...]
    h = jnp.dot(h, w1_ref[...], preferred_element_type=jnp.float32)
    h = jnp.maximum(h + b1_ref[...], 0.0)
    h = (h - m1_ref[...]) * lax.rsqrt(v1_ref[...] + EPS) * g1_ref[...] + be1_ref[...]
    out_ref[...] = jnp.dot(h, w2_ref[...], preferred_element_type=jnp.float32) + b2_ref[...]


def _tc_mlp(xc, xn, bg, bb, bm, bv, w0c, w0n, b0, g0, be0, m0, v0,
            w1, b1, g1, be1, m1, v1, w2, b2):
    TB = 512
    grid = (B // TB,)
    row = lambda i: (i, 0)
    rep = lambda i: (0, 0)
    full = lambda a: pl.BlockSpec(a.shape, rep)
    return pl.pallas_call(
        _mlp_body,
        grid=grid,
        in_specs=[
            pl.BlockSpec((TB, F * D), row),
            pl.BlockSpec((TB, NUM), row),
            full(bg), full(bb), full(bm), full(bv),
            full(w0c), full(w0n), full(b0), full(g0), full(be0), full(m0), full(v0),
            full(w1), full(b1), full(g1), full(be1), full(m1), full(v1),
            full(w2), full(b2),
        ],
        out_specs=pl.BlockSpec((TB, 1), row),
        out_shape=jax.ShapeDtypeStruct((B, 1), jnp.float32),
    )(xc, xn, bg, bb, bm, bv, w0c, w0n, b0, g0, be0, m0, v0,
      w1, b1, g1, be1, m1, v1, w2, b2)


def kernel(x_categorical, x_numerical, emb_tables, bn_num_gamma, bn_num_beta,
           bn_num_mean, bn_num_var, w0, b0, g0, be0, m0, v0,
           w1, b1, g1, be1, m1, v1, w2, b2):
    v = x_categorical.astype(jnp.int32)
    e = ((jnp.arange(F, dtype=jnp.int32) * V)[None, :] + v).reshape(-1)
    table2 = jnp.swapaxes(emb_tables, 1, 2).reshape(F * D, V)
    staged = _tc_transpose(table2)
    tail = lax.slice(table2, (0, NJ * BV), (F * D, V))
    tail_t = jnp.swapaxes(tail.reshape(F, D, VTAIL), 1, 2).reshape(F, VTAIL * D // 128, 128)
    staged = lax.dynamic_update_slice(staged, tail_t, (0, NJ * BV * D // 128, 0))
    s128 = staged.reshape(SROWS, 128)
    rows = _sc_gather(s128, e)
    xc = rows.reshape(B, F * D)

    r2 = lambda a: a.reshape(1, -1)
    return _tc_mlp(
        xc, x_numerical,
        r2(bn_num_gamma), r2(bn_num_beta), r2(bn_num_mean), r2(bn_num_var),
        w0[:, :F * D].T, w0[:, F * D:].T, r2(b0), r2(g0), r2(be0), r2(m0), r2(v0),
        w1.T, r2(b1), r2(g1), r2(be1), r2(m1), r2(v1),
        w2.T, r2(b2),
    )


# speedup vs baseline: 4.2607x; 1.0320x over previous
import functools

import jax
import jax.numpy as jnp
from jax import lax
from jax.experimental import pallas as pl
from jax.experimental.pallas import tpu as pltpu
from jax.experimental.pallas import tpu_sc as plsc

B = 4096
F = 26
V = 1000000
D = 16
NUM = 13
H1 = 256
H2 = 128
EPS = 1e-5

NC = 2
NS = 16
NW = NC * NS
R = B * F
RPW = R // NW
CH = 128
NCH = RPW // CH
SROWS = F * V * D // 128

_mesh = plsc.VectorSubcoreMesh(core_axis_name="c", subcore_axis_name="s")


BV = 7936
NJ = 126
NSTEPS = F * NJ
VTAIL = V - NJ * BV


def _tr_body(x_hbm, o_ref, buf, sem):
    f = pl.program_id(0)
    j = pl.program_id(1)
    k = f * NJ + j

    @pl.when(k == 0)
    def _():
        pltpu.make_async_copy(
            x_hbm.at[pl.ds(0, D), pl.ds(0, BV)], buf.at[0], sem).start()

    @pl.when(k + 1 < NSTEPS)
    def _():
        k1 = k + 1
        nf = k1 // NJ
        nj = k1 % NJ
        pltpu.make_async_copy(
            x_hbm.at[pl.ds(pl.multiple_of(nf * D, 8), D),
                     pl.ds(pl.multiple_of(nj * BV, 128), BV)],
            buf.at[lax.rem(k1, 2)], sem).start()

    pltpu.make_async_copy(
        x_hbm.at[pl.ds(0, D), pl.ds(0, BV)], buf.at[lax.rem(k, 2)], sem).wait()
    x = buf[lax.rem(k, 2)]
    eye = (lax.broadcasted_iota(jnp.int32, (D, D), 0)
           == lax.broadcasted_iota(jnp.int32, (D, D), 1)).astype(jnp.float32)
    y = lax.dot_general(x, eye, (((0,), (0,)), ((), ())),
                        preferred_element_type=jnp.float32)
    y8 = y.reshape(BV // 8, 8, D)
    lc = lax.broadcasted_iota(jnp.int32, (D, 128), 1)
    ld = lax.broadcasted_iota(jnp.int32, (D, 128), 0)
    o = None
    for m in range(8):
        em = (lc == ld + m * D).astype(jnp.float32)
        part = jnp.dot(y8[:, m, :], em, preferred_element_type=jnp.float32)
        o = part if o is None else o + part
    o_ref[...] = o.reshape(1, BV // 8, 128)


def _tc_transpose(table2):
    grid = (F, NJ)
    return pl.pallas_call(
        _tr_body,
        grid=grid,
        in_specs=[pl.BlockSpec(memory_space=pl.ANY)],
        out_specs=pl.BlockSpec((1, BV // 8, 128), lambda f, j: (f, j, 0)),
        out_shape=jax.ShapeDtypeStruct((F, V * D // 128, 128), jnp.float32),
        scratch_shapes=[
            pltpu.VMEM((2, D, BV), jnp.float32),
            pltpu.SemaphoreType.DMA,
        ],
    )(table2)


@functools.partial(
    pl.kernel,
    mesh=_mesh,
    out_type=jax.ShapeDtypeStruct((R * D,), jnp.float32),
    scratch_types=[
        pltpu.VMEM((RPW,), jnp.int32),
        pltpu.VMEM((2, CH), jnp.int32),
        pltpu.VMEM((2, CH, 128), jnp.float32),
        pltpu.VMEM((RPW * D,), jnp.float32),
        pltpu.SemaphoreType.DMA,
    ],
    compiler_params=pltpu.CompilerParams(
        use_tc_tiling_on_sc=False, needs_layout_passes=False),
)
def _sc_gather(table_hbm, e_hbm, out_hbm, e_v, q_v, ring_v, rows_v, sem):
    wid = lax.axis_index("s") * NC + lax.axis_index("c")
    base = wid * RPW
    pltpu.sync_copy(e_hbm.at[pl.ds(base, RPW)], e_v)
    iota = lax.iota(jnp.int32, 16)

    def fire(g):
        slot = lax.rem(g, 2)
        for j in range(CH // 16):
            q16 = e_v[pl.ds(g * CH + j * 16, 16)] >> 3
            q_v[slot, pl.ds(j * 16, 16)] = q16
        pltpu.make_async_copy(
            table_hbm.at[q_v.at[slot]], ring_v.at[slot], sem).start()

    def drain(g):
        slot = lax.rem(g, 2)
        pltpu.make_async_copy(
            table_hbm.at[q_v.at[slot]], ring_v.at[slot], sem).wait()
        for j in range(CH // 16):
            m16 = (e_v[pl.ds(g * CH + j * 16, 16)] & 7) * 16
            for t in range(16):
                jj = j * 16 + t
                vals = plsc.load_gather(
                    ring_v, [jnp.broadcast_to(slot, (16,)),
                             jnp.broadcast_to(jj, (16,)),
                             m16[t] + iota])
                plsc.store_scatter(rows_v, [(g * CH + jj) * D + iota], vals)

    fire(0)

    def body(g, _):
        fire(g)
        drain(g - 1)
        return 0

    lax.fori_loop(1, NCH, body, 0)
    drain(NCH - 1)
    pltpu.sync_copy(rows_v, out_hbm.at[pl.ds(base * D, RPW * D)])


def _mlp_body(xc_ref, xn_ref,
              bg_ref, bb_ref, bm_ref, bv_ref,
              w0c_ref, w0n_ref, b0_ref, g0_ref, be0_ref, m0_ref, v0_ref,
              w1_ref, b1_ref, g1_ref, be1_ref, m1_ref, v1_ref,
              w2_ref, b2_ref, out_ref):
    xn = xn_ref[...]
    xnb = (xn - bm_ref[...]) * lax.rsqrt(bv_ref[...] + EPS) * bg_ref[...] + bb_ref[...]
    h = jnp.dot(xc_ref[...], w0c_ref[...], preferred_element_type=jnp.float32)
    h = h + jnp.dot(xnb, w0n_ref[...], preferred_element_type=jnp.float32)
    h = jnp.maximum(h + b0_ref[...], 0.0)
    h = (h - m0_ref[...]) * lax.rsqrt(v0_ref[...] + EPS) * g0_ref[...] + be0_ref[...]
    h = jnp.dot(h, w1_ref[...], preferred_element_type=jnp.float32)
    h = jnp.maximum(h + b1_ref[...], 0.0)
    h = (h - m1_ref[...]) * lax.rsqrt(v1_ref[...] + EPS) * g1_ref[...] + be1_ref[...]
    out_ref[...] = jnp.dot(h, w2_ref[...], preferred_element_type=jnp.float32) + b2_ref[...]


def _tc_mlp(xc, xn, bg, bb, bm, bv, w0c, w0n, b0, g0, be0, m0, v0,
            w1, b1, g1, be1, m1, v1, w2, b2):
    TB = 512
    grid = (B // TB,)
    row = lambda i: (i, 0)
    rep = lambda i: (0, 0)
    full = lambda a: pl.BlockSpec(a.shape, rep)
    return pl.pallas_call(
        _mlp_body,
        grid=grid,
        in_specs=[
            pl.BlockSpec((TB, F * D), row),
            pl.BlockSpec((TB, NUM), row),
            full(bg), full(bb), full(bm), full(bv),
            full(w0c), full(w0n), full(b0), full(g0), full(be0), full(m0), full(v0),
            full(w1), full(b1), full(g1), full(be1), full(m1), full(v1),
            full(w2), full(b2),
        ],
        out_specs=pl.BlockSpec((TB, 1), row),
        out_shape=jax.ShapeDtypeStruct((B, 1), jnp.float32),
    )(xc, xn, bg, bb, bm, bv, w0c, w0n, b0, g0, be0, m0, v0,
      w1, b1, g1, be1, m1, v1, w2, b2)


def kernel(x_categorical, x_numerical, emb_tables, bn_num_gamma, bn_num_beta,
           bn_num_mean, bn_num_var, w0, b0, g0, be0, m0, v0,
           w1, b1, g1, be1, m1, v1, w2, b2):
    v = x_categorical.astype(jnp.int32)
    e = ((jnp.arange(F, dtype=jnp.int32) * V)[None, :] + v).reshape(-1)
    table2 = jnp.swapaxes(emb_tables, 1, 2).reshape(F * D, V)
    staged = _tc_transpose(table2)
    tail = lax.slice(table2, (0, NJ * BV), (F * D, V))
    tail_t = jnp.swapaxes(tail.reshape(F, D, VTAIL), 1, 2).reshape(F, VTAIL * D // 128, 128)
    staged = lax.dynamic_update_slice(staged, tail_t, (0, NJ * BV * D // 128, 0))
    s128 = staged.reshape(SROWS, 128)
    rows = _sc_gather(s128, e)
    xc = rows.reshape(B, F * D)

    r2 = lambda a: a.reshape(1, -1)
    return _tc_mlp(
        xc, x_numerical,
        r2(bn_num_gamma), r2(bn_num_beta), r2(bn_num_mean), r2(bn_num_var),
        w0[:, :F * D].T, w0[:, F * D:].T, r2(b0), r2(g0), r2(be0), r2(m0), r2(v0),
        w1.T, r2(b1), r2(g1), r2(be1), r2(m1), r2(v1),
        w2.T, r2(b2),
    )


# speedup vs baseline: 7.0480x; 1.6542x over previous
import jax
import jax.numpy as jnp
from jax import lax
from jax.experimental import pallas as pl
from jax.experimental.pallas import tpu as pltpu

B = 4096
F = 26
V = 1000000
D = 16
NUM = 13
H1 = 256
H2 = 128
EPS = 1e-5

R = B * F
LPS = 16
NSTEP = R // LPS


def _gather_body(pk_ref, *refs):
    slab_refs = refs[:LPS]
    out_ref = refs[LPS]
    i = pl.program_id(0)
    S = jnp.concatenate([s[...] for s in slab_refs], axis=0)
    eye = (lax.broadcasted_iota(jnp.int32, (256, 256), 0)
           == lax.broadcasted_iota(jnp.int32, (256, 256), 1)).astype(jnp.float32)
    st = lax.dot_general(S, eye, (((0,), (0,)), ((), ())),
                         preferred_element_type=jnp.float32)
    rows = []
    for a in range(2):
        lcol = jnp.concatenate(
            [jnp.full((D,), pk_ref[i * LPS + a * 8 + j] & 127, jnp.int32)
             for j in range(8)])
        blk = st[:, a * 128:(a + 1) * 128]
        mask = (lax.broadcasted_iota(jnp.int32, (128, 128), 0)
                == lcol.reshape(1, 128)).astype(jnp.float32)
        rows.append(jnp.sum(blk * mask, axis=0))
    out_ref[...] = jnp.stack(rows, axis=0).reshape(1, 2, 128)


def _tc_gather(table2, pk):
    slab_spec = [
        pl.BlockSpec((D, 128),
                     (lambda j: (lambda i, pk_r:
                                 (pk_r[i * LPS + j] >> 20,
                                  (pk_r[i * LPS + j] >> 7) & 8191)))(j))
        for j in range(LPS)
    ]
    grid_spec = pltpu.PrefetchScalarGridSpec(
        num_scalar_prefetch=1,
        grid=(NSTEP,),
        in_specs=slab_spec,
        out_specs=pl.BlockSpec((1, 2, 128), lambda i, pk_r: (i, 0, 0)),
    )
    return pl.pallas_call(
        _gather_body,
        grid_spec=grid_spec,
        out_shape=jax.ShapeDtypeStruct((NSTEP, 2, 128), jnp.float32),
    )(pk, *([table2] * LPS))


def _mlp_body(xc_ref, xn_ref,
              bg_ref, bb_ref, bm_ref, bv_ref,
              w0c_ref, w0n_ref, b0_ref, g0_ref, be0_ref, m0_ref, v0_ref,
              w1_ref, b1_ref, g1_ref, be1_ref, m1_ref, v1_ref,
              w2_ref, b2_ref, out_ref):
    xn = xn_ref[...]
    xnb = (xn - bm_ref[...]) * lax.rsqrt(bv_ref[...] + EPS) * bg_ref[...] + bb_ref[...]
    h = jnp.dot(xc_ref[...], w0c_ref[...], preferred_element_type=jnp.float32)
    h = h + jnp.dot(xnb, w0n_ref[...], preferred_element_type=jnp.float32)
    h = jnp.maximum(h + b0_ref[...], 0.0)
    h = (h - m0_ref[...]) * lax.rsqrt(v0_ref[...] + EPS) * g0_ref[...] + be0_ref[...]
    h = jnp.dot(h, w1_ref[...], preferred_element_type=jnp.float32)
    h = jnp.maximum(h + b1_ref[...], 0.0)
    h = (h - m1_ref[...]) * lax.rsqrt(v1_ref[...] + EPS) * g1_ref[...] + be1_ref[...]
    out_ref[...] = jnp.dot(h, w2_ref[...], preferred_element_type=jnp.float32) + b2_ref[...]


def _tc_mlp(xc, xn, bg, bb, bm, bv, w0c, w0n, b0, g0, be0, m0, v0,
            w1, b1, g1, be1, m1, v1, w2, b2):
    TB = 512
    grid = (B // TB,)
    row = lambda i: (i, 0)
    rep = lambda i: (0, 0)
    full = lambda a: pl.BlockSpec(a.shape, rep)
    return pl.pallas_call(
        _mlp_body,
        grid=grid,
        in_specs=[
            pl.BlockSpec((TB, F * D), row),
            pl.BlockSpec((TB, NUM), row),
            full(bg), full(bb), full(bm), full(bv),
            full(w0c), full(w0n), full(b0), full(g0), full(be0), full(m0), full(v0),
            full(w1), full(b1), full(g1), full(be1), full(m1), full(v1),
            full(w2), full(b2),
        ],
        out_specs=pl.BlockSpec((TB, 1), row),
        out_shape=jax.ShapeDtypeStruct((B, 1), jnp.float32),
    )(xc, xn, bg, bb, bm, bv, w0c, w0n, b0, g0, be0, m0, v0,
      w1, b1, g1, be1, m1, v1, w2, b2)


def kernel(x_categorical, x_numerical, emb_tables, bn_num_gamma, bn_num_beta,
           bn_num_mean, bn_num_var, w0, b0, g0, be0, m0, v0,
           w1, b1, g1, be1, m1, v1, w2, b2):
    v = x_categorical.astype(jnp.int32)
    fcol = jnp.arange(F, dtype=jnp.int32)[None, :]
    pk = ((fcol << 20) | (((v >> 7) & 8191) << 7) | (v & 127)).reshape(-1)
    table2 = jnp.swapaxes(emb_tables, 1, 2).reshape(F * D, V)
    g = _tc_gather(table2, pk)
    xc = g.reshape(B, F * D)

    r2 = lambda a: a.reshape(1, -1)
    return _tc_mlp(
        xc, x_numerical,
        r2(bn_num_gamma), r2(bn_num_beta), r2(bn_num_mean), r2(bn_num_var),
        w0[:, :F * D].T, w0[:, F * D:].T, r2(b0), r2(g0), r2(be0), r2(m0), r2(v0),
        w1.T, r2(b1), r2(g1), r2(be1), r2(m1), r2(v1),
        w2.T, r2(b2),
    )


# speedup vs baseline: 9.1037x; 1.2917x over previous
import jax
import jax.numpy as jnp
from jax import lax
from jax.experimental import pallas as pl
from jax.experimental.pallas import tpu as pltpu

B = 4096
F = 26
V = 1000000
D = 16
NUM = 13
H1 = 256
H2 = 128
EPS = 1e-5

R = B * F
LPS = 32
NSTEP = R // LPS


def _gather_body(pk_ref, *refs):
    slab_refs = refs[:LPS]
    out_ref = refs[LPS]
    i = pl.program_id(0)
    S = jnp.concatenate([s[...] for s in slab_refs], axis=0)
    KK = LPS * D
    eye = (lax.broadcasted_iota(jnp.int32, (KK, KK), 0)
           == lax.broadcasted_iota(jnp.int32, (KK, KK), 1)).astype(jnp.float32)
    st = lax.dot_general(S, eye, (((0,), (0,)), ((), ())),
                         preferred_element_type=jnp.float32)
    rows = []
    for a in range(LPS // 8):
        lcol = jnp.concatenate(
            [jnp.full((D,), pk_ref[i * LPS + a * 8 + j] & 127, jnp.int32)
             for j in range(8)])
        blk = st[:, a * 128:(a + 1) * 128]
        mask = (lax.broadcasted_iota(jnp.int32, (128, 128), 0)
                == lcol.reshape(1, 128)).astype(jnp.float32)
        rows.append(jnp.sum(blk * mask, axis=0))
    out_ref[...] = jnp.stack(rows, axis=0).reshape(1, LPS // 8, 128)


def _tc_gather(table2, pk):
    slab_spec = [
        pl.BlockSpec((D, 128),
                     (lambda j: (lambda i, pk_r:
                                 (pk_r[i * LPS + j] >> 20,
                                  (pk_r[i * LPS + j] >> 7) & 8191)))(j))
        for j in range(LPS)
    ]
    grid_spec = pltpu.PrefetchScalarGridSpec(
        num_scalar_prefetch=1,
        grid=(NSTEP,),
        in_specs=slab_spec,
        out_specs=pl.BlockSpec((1, LPS // 8, 128), lambda i, pk_r: (i, 0, 0)),
    )
    return pl.pallas_call(
        _gather_body,
        grid_spec=grid_spec,
        out_shape=jax.ShapeDtypeStruct((NSTEP, LPS // 8, 128), jnp.float32),
    )(pk, *([table2] * LPS))


def _mlp_body(xc_ref, xn_ref,
              bg_ref, bb_ref, bm_ref, bv_ref,
              w0c_ref, w0n_ref, b0_ref, g0_ref, be0_ref, m0_ref, v0_ref,
              w1_ref, b1_ref, g1_ref, be1_ref, m1_ref, v1_ref,
              w2_ref, b2_ref, out_ref):
    xn = xn_ref[...]
    xnb = (xn - bm_ref[...]) * lax.rsqrt(bv_ref[...] + EPS) * bg_ref[...] + bb_ref[...]
    h = jnp.dot(xc_ref[...], w0c_ref[...], preferred_element_type=jnp.float32)
    h = h + jnp.dot(xnb, w0n_ref[...], preferred_element_type=jnp.float32)
    h = jnp.maximum(h + b0_ref[...], 0.0)
    h = (h - m0_ref[...]) * lax.rsqrt(v0_ref[...] + EPS) * g0_ref[...] + be0_ref[...]
    h = jnp.dot(h, w1_ref[...], preferred_element_type=jnp.float32)
    h = jnp.maximum(h + b1_ref[...], 0.0)
    h = (h - m1_ref[...]) * lax.rsqrt(v1_ref[...] + EPS) * g1_ref[...] + be1_ref[...]
    out_ref[...] = jnp.dot(h, w2_ref[...], preferred_element_type=jnp.float32) + b2_ref[...]


def _tc_mlp(xc, xn, bg, bb, bm, bv, w0c, w0n, b0, g0, be0, m0, v0,
            w1, b1, g1, be1, m1, v1, w2, b2):
    TB = 512
    grid = (B // TB,)
    row = lambda i: (i, 0)
    rep = lambda i: (0, 0)
    full = lambda a: pl.BlockSpec(a.shape, rep)
    return pl.pallas_call(
        _mlp_body,
        grid=grid,
        in_specs=[
            pl.BlockSpec((TB, F * D), row),
            pl.BlockSpec((TB, NUM), row),
            full(bg), full(bb), full(bm), full(bv),
            full(w0c), full(w0n), full(b0), full(g0), full(be0), full(m0), full(v0),
            full(w1), full(b1), full(g1), full(be1), full(m1), full(v1),
            full(w2), full(b2),
        ],
        out_specs=pl.BlockSpec((TB, 1), row),
        out_shape=jax.ShapeDtypeStruct((B, 1), jnp.float32),
    )(xc, xn, bg, bb, bm, bv, w0c, w0n, b0, g0, be0, m0, v0,
      w1, b1, g1, be1, m1, v1, w2, b2)


def kernel(x_categorical, x_numerical, emb_tables, bn_num_gamma, bn_num_beta,
           bn_num_mean, bn_num_var, w0, b0, g0, be0, m0, v0,
           w1, b1, g1, be1, m1, v1, w2, b2):
    v = x_categorical.astype(jnp.int32)
    fcol = jnp.arange(F, dtype=jnp.int32)[None, :]
    pk = ((fcol << 20) | (((v >> 7) & 8191) << 7) | (v & 127)).reshape(-1)
    table2 = jnp.swapaxes(emb_tables, 1, 2).reshape(F * D, V)
    g = _tc_gather(table2, pk)
    xc = g.reshape(B, F * D)

    r2 = lambda a: a.reshape(1, -1)
    return _tc_mlp(
        xc, x_numerical,
        r2(bn_num_gamma), r2(bn_num_beta), r2(bn_num_mean), r2(bn_num_var),
        w0[:, :F * D].T, w0[:, F * D:].T, r2(b0), r2(g0), r2(be0), r2(m0), r2(v0),
        w1.T, r2(b1), r2(g1), r2(be1), r2(m1), r2(v1),
        w2.T, r2(b2),
    )


# speedup vs baseline: 9.9507x; 1.0930x over previous
import jax
import jax.numpy as jnp
from jax import lax
from jax.experimental import pallas as pl
from jax.experimental.pallas import tpu as pltpu

B = 4096
F = 26
V = 1000000
D = 16
NUM = 13
H1 = 256
H2 = 128
EPS = 1e-5

R = B * F
LPS = 32
NSTEP = R // LPS


def _gather_body(pk_ref, *refs):
    slab_refs = refs[:LPS]
    out_ref = refs[LPS]
    i = pl.program_id(0)
    eye = (lax.broadcasted_iota(jnp.int32, (128, 128), 0)
           == lax.broadcasted_iota(jnp.int32, (128, 128), 1)).astype(jnp.float32)
    rows = []
    for a in range(LPS // 8):
        sa = jnp.concatenate(
            [s[...] for s in slab_refs[a * 8:(a + 1) * 8]], axis=0)
        blk = lax.dot_general(sa, eye, (((0,), (0,)), ((), ())),
                              preferred_element_type=jnp.float32)
        lcol = jnp.concatenate(
            [jnp.full((D,), pk_ref[i * LPS + a * 8 + j] & 127, jnp.int32)
             for j in range(8)])
        mask = (lax.broadcasted_iota(jnp.int32, (128, 128), 0)
                == lcol.reshape(1, 128)).astype(jnp.float32)
        rows.append(jnp.sum(blk * mask, axis=0))
    out_ref[...] = jnp.stack(rows, axis=0).reshape(1, LPS // 8, 128)


def _tc_gather(table2, pk):
    slab_spec = [
        pl.BlockSpec((D, 128),
                     (lambda j: (lambda i, pk_r:
                                 (pk_r[i * LPS + j] >> 20,
                                  (pk_r[i * LPS + j] >> 7) & 8191)))(j))
        for j in range(LPS)
    ]
    grid_spec = pltpu.PrefetchScalarGridSpec(
        num_scalar_prefetch=1,
        grid=(NSTEP,),
        in_specs=slab_spec,
        out_specs=pl.BlockSpec((1, LPS // 8, 128), lambda i, pk_r: (i, 0, 0)),
    )
    return pl.pallas_call(
        _gather_body,
        grid_spec=grid_spec,
        out_shape=jax.ShapeDtypeStruct((NSTEP, LPS // 8, 128), jnp.float32),
    )(pk, *([table2] * LPS))


def _mlp_body(xc_ref, xn_ref,
              bg_ref, bb_ref, bm_ref, bv_ref,
              w0c_ref, w0n_ref, b0_ref, g0_ref, be0_ref, m0_ref, v0_ref,
              w1_ref, b1_ref, g1_ref, be1_ref, m1_ref, v1_ref,
              w2_ref, b2_ref, out_ref):
    xn = xn_ref[...]
    xnb = (xn - bm_ref[...]) * lax.rsqrt(bv_ref[...] + EPS) * bg_ref[...] + bb_ref[...]
    h = jnp.dot(xc_ref[...], w0c_ref[...], preferred_element_type=jnp.float32)
    h = h + jnp.dot(xnb, w0n_ref[...], preferred_element_type=jnp.float32)
    h = jnp.maximum(h + b0_ref[...], 0.0)
    h = (h - m0_ref[...]) * lax.rsqrt(v0_ref[...] + EPS) * g0_ref[...] + be0_ref[...]
    h = jnp.dot(h, w1_ref[...], preferred_element_type=jnp.float32)
    h = jnp.maximum(h + b1_ref[...], 0.0)
    h = (h - m1_ref[...]) * lax.rsqrt(v1_ref[...] + EPS) * g1_ref[...] + be1_ref[...]
    out_ref[...] = jnp.dot(h, w2_ref[...], preferred_element_type=jnp.float32) + b2_ref[...]


def _tc_mlp(xc, xn, bg, bb, bm, bv, w0c, w0n, b0, g0, be0, m0, v0,
            w1, b1, g1, be1, m1, v1, w2, b2):
    TB = 512
    grid = (B // TB,)
    row = lambda i: (i, 0)
    rep = lambda i: (0, 0)
    full = lambda a: pl.BlockSpec(a.shape, rep)
    return pl.pallas_call(
        _mlp_body,
        grid=grid,
        in_specs=[
            pl.BlockSpec((TB, F * D), row),
            pl.BlockSpec((TB, NUM), row),
            full(bg), full(bb), full(bm), full(bv),
            full(w0c), full(w0n), full(b0), full(g0), full(be0), full(m0), full(v0),
            full(w1), full(b1), full(g1), full(be1), full(m1), full(v1),
            full(w2), full(b2),
        ],
        out_specs=pl.BlockSpec((TB, 1), row),
        out_shape=jax.ShapeDtypeStruct((B, 1), jnp.float32),
    )(xc, xn, bg, bb, bm, bv, w0c, w0n, b0, g0, be0, m0, v0,
      w1, b1, g1, be1, m1, v1, w2, b2)


def kernel(x_categorical, x_numerical, emb_tables, bn_num_gamma, bn_num_beta,
           bn_num_mean, bn_num_var, w0, b0, g0, be0, m0, v0,
           w1, b1, g1, be1, m1, v1, w2, b2):
    v = x_categorical.astype(jnp.int32)
    fcol = jnp.arange(F, dtype=jnp.int32)[None, :]
    pk = ((fcol << 20) | (((v >> 7) & 8191) << 7) | (v & 127)).reshape(-1)
    table2 = jnp.swapaxes(emb_tables, 1, 2).reshape(F * D, V)
    g = _tc_gather(table2, pk)
    xc = g.reshape(B, F * D)

    r2 = lambda a: a.reshape(1, -1)
    return _tc_mlp(
        xc, x_numerical,
        r2(bn_num_gamma), r2(bn_num_beta), r2(bn_num_mean), r2(bn_num_var),
        w0[:, :F * D].T, w0[:, F * D:].T, r2(b0), r2(g0), r2(be0), r2(m0), r2(v0),
        w1.T, r2(b1), r2(g1), r2(be1), r2(m1), r2(v1),
        w2.T, r2(b2),
    )


# speedup vs baseline: 11.0747x; 1.1130x over previous
import jax
import jax.numpy as jnp
from jax import lax
from jax.experimental import pallas as pl
from jax.experimental.pallas import tpu as pltpu

B = 4096
F = 26
V = 1000000
D = 16
NUM = 13
H1 = 256
H2 = 128
EPS = 1e-5

R = B * F
LPS = 64
NSTEP = R // LPS


def _gather_body(pk_ref, *refs):
    slab_refs = refs[:LPS]
    out_ref = refs[LPS]
    i = pl.program_id(0)
    eye = (lax.broadcasted_iota(jnp.int32, (128, 128), 0)
           == lax.broadcasted_iota(jnp.int32, (128, 128), 1)).astype(jnp.float32)
    rows = []
    for a in range(LPS // 8):
        sa = jnp.concatenate(
            [s[...] for s in slab_refs[a * 8:(a + 1) * 8]], axis=0)
        blk = lax.dot_general(sa, eye, (((0,), (0,)), ((), ())),
                              preferred_element_type=jnp.float32)
        lcol = jnp.concatenate(
            [jnp.full((D,), pk_ref[i * LPS + a * 8 + j] & 127, jnp.int32)
             for j in range(8)])
        mask = (lax.broadcasted_iota(jnp.int32, (128, 128), 0)
                == lcol.reshape(1, 128)).astype(jnp.float32)
        rows.append(jnp.sum(blk * mask, axis=0))
    out_ref[...] = jnp.stack(rows, axis=0).reshape(1, LPS // 8, 128)


def _tc_gather(table2, pk):
    slab_spec = [
        pl.BlockSpec((D, 128),
                     (lambda j: (lambda i, pk_r:
                                 (pk_r[i * LPS + j] >> 20,
                                  (pk_r[i * LPS + j] >> 7) & 8191)))(j))
        for j in range(LPS)
    ]
    grid_spec = pltpu.PrefetchScalarGridSpec(
        num_scalar_prefetch=1,
        grid=(NSTEP,),
        in_specs=slab_spec,
        out_specs=pl.BlockSpec((1, LPS // 8, 128), lambda i, pk_r: (i, 0, 0)),
    )
    return pl.pallas_call(
        _gather_body,
        grid_spec=grid_spec,
        out_shape=jax.ShapeDtypeStruct((NSTEP, LPS // 8, 128), jnp.float32),
    )(pk, *([table2] * LPS))


def _mlp_body(xc_ref, xn_ref,
              bg_ref, bb_ref, bm_ref, bv_ref,
              w0c_ref, w0n_ref, b0_ref, g0_ref, be0_ref, m0_ref, v0_ref,
              w1_ref, b1_ref, g1_ref, be1_ref, m1_ref, v1_ref,
              w2_ref, b2_ref, out_ref):
    xn = xn_ref[...]
    xnb = (xn - bm_ref[...]) * lax.rsqrt(bv_ref[...] + EPS) * bg_ref[...] + bb_ref[...]
    h = jnp.dot(xc_ref[...], w0c_ref[...], preferred_element_type=jnp.float32)
    h = h + jnp.dot(xnb, w0n_ref[...], preferred_element_type=jnp.float32)
    h = jnp.maximum(h + b0_ref[...], 0.0)
    h = (h - m0_ref[...]) * lax.rsqrt(v0_ref[...] + EPS) * g0_ref[...] + be0_ref[...]
    h = jnp.dot(h, w1_ref[...], preferred_element_type=jnp.float32)
    h = jnp.maximum(h + b1_ref[...], 0.0)
    h = (h - m1_ref[...]) * lax.rsqrt(v1_ref[...] + EPS) * g1_ref[...] + be1_ref[...]
    out_ref[...] = jnp.dot(h, w2_ref[...], preferred_element_type=jnp.float32) + b2_ref[...]


def _tc_mlp(xc, xn, bg, bb, bm, bv, w0c, w0n, b0, g0, be0, m0, v0,
            w1, b1, g1, be1, m1, v1, w2, b2):
    TB = 512
    grid = (B // TB,)
    row = lambda i: (i, 0)
    rep = lambda i: (0, 0)
    full = lambda a: pl.BlockSpec(a.shape, rep)
    return pl.pallas_call(
        _mlp_body,
        grid=grid,
        in_specs=[
            pl.BlockSpec((TB, F * D), row),
            pl.BlockSpec((TB, NUM), row),
            full(bg), full(bb), full(bm), full(bv),
            full(w0c), full(w0n), full(b0), full(g0), full(be0), full(m0), full(v0),
            full(w1), full(b1), full(g1), full(be1), full(m1), full(v1),
            full(w2), full(b2),
        ],
        out_specs=pl.BlockSpec((TB, 1), row),
        out_shape=jax.ShapeDtypeStruct((B, 1), jnp.float32),
    )(xc, xn, bg, bb, bm, bv, w0c, w0n, b0, g0, be0, m0, v0,
      w1, b1, g1, be1, m1, v1, w2, b2)


def kernel(x_categorical, x_numerical, emb_tables, bn_num_gamma, bn_num_beta,
           bn_num_mean, bn_num_var, w0, b0, g0, be0, m0, v0,
           w1, b1, g1, be1, m1, v1, w2, b2):
    v = x_categorical.astype(jnp.int32)
    fcol = jnp.arange(F, dtype=jnp.int32)[None, :]
    pk = ((fcol << 20) | (((v >> 7) & 8191) << 7) | (v & 127)).reshape(-1)
    table2 = jnp.swapaxes(emb_tables, 1, 2).reshape(F * D, V)
    g = _tc_gather(table2, pk)
    xc = g.reshape(B, F * D)

    r2 = lambda a: a.reshape(1, -1)
    return _tc_mlp(
        xc, x_numerical,
        r2(bn_num_gamma), r2(bn_num_beta), r2(bn_num_mean), r2(bn_num_var),
        w0[:, :F * D].T, w0[:, F * D:].T, r2(b0), r2(g0), r2(be0), r2(m0), r2(v0),
        w1.T, r2(b1), r2(g1), r2(be1), r2(m1), r2(v1),
        w2.T, r2(b2),
    )


# speedup vs baseline: 11.7938x; 1.0649x over previous
import jax
import jax.numpy as jnp
from jax import lax
from jax.experimental import pallas as pl
from jax.experimental.pallas import tpu as pltpu

B = 4096
F = 26
V = 1000000
D = 16
NUM = 13
H1 = 256
H2 = 128
EPS = 1e-5

R = B * F
LPS = 128
NSTEP = R // LPS


def _gather_body(pk_ref, *refs):
    slab_refs = refs[:LPS]
    out_ref = refs[LPS]
    i = pl.program_id(0)
    eye = (lax.broadcasted_iota(jnp.int32, (128, 128), 0)
           == lax.broadcasted_iota(jnp.int32, (128, 128), 1)).astype(jnp.float32)
    rows = []
    for a in range(LPS // 8):
        sa = jnp.concatenate(
            [s[...] for s in slab_refs[a * 8:(a + 1) * 8]], axis=0)
        blk = lax.dot_general(sa, eye, (((0,), (0,)), ((), ())),
                              preferred_element_type=jnp.float32)
        lcol = jnp.concatenate(
            [jnp.full((D,), pk_ref[i * LPS + a * 8 + j] & 127, jnp.int32)
             for j in range(8)])
        mask = (lax.broadcasted_iota(jnp.int32, (128, 128), 0)
                == lcol.reshape(1, 128)).astype(jnp.float32)
        rows.append(jnp.sum(blk * mask, axis=0))
    out_ref[...] = jnp.stack(rows, axis=0).reshape(1, LPS // 8, 128)


def _tc_gather(table2, pk):
    slab_spec = [
        pl.BlockSpec((D, 128),
                     (lambda j: (lambda i, pk_r:
                                 (pk_r[i * LPS + j] >> 20,
                                  (pk_r[i * LPS + j] >> 7) & 8191)))(j))
        for j in range(LPS)
    ]
    grid_spec = pltpu.PrefetchScalarGridSpec(
        num_scalar_prefetch=1,
        grid=(NSTEP,),
        in_specs=slab_spec,
        out_specs=pl.BlockSpec((1, LPS // 8, 128), lambda i, pk_r: (i, 0, 0)),
    )
    return pl.pallas_call(
        _gather_body,
        grid_spec=grid_spec,
        out_shape=jax.ShapeDtypeStruct((NSTEP, LPS // 8, 128), jnp.float32),
    )(pk, *([table2] * LPS))


def _mlp_body(xc_ref, xn_ref,
              bg_ref, bb_ref, bm_ref, bv_ref,
              w0c_ref, w0n_ref, b0_ref, g0_ref, be0_ref, m0_ref, v0_ref,
              w1_ref, b1_ref, g1_ref, be1_ref, m1_ref, v1_ref,
              w2_ref, b2_ref, out_ref):
    xn = xn_ref[...]
    xnb = (xn - bm_ref[...]) * lax.rsqrt(bv_ref[...] + EPS) * bg_ref[...] + bb_ref[...]
    h = jnp.dot(xc_ref[...], w0c_ref[...], preferred_element_type=jnp.float32)
    h = h + jnp.dot(xnb, w0n_ref[...], preferred_element_type=jnp.float32)
    h = jnp.maximum(h + b0_ref[...], 0.0)
    h = (h - m0_ref[...]) * lax.rsqrt(v0_ref[...] + EPS) * g0_ref[...] + be0_ref[...]
    h = jnp.dot(h, w1_ref[...], preferred_element_type=jnp.float32)
    h = jnp.maximum(h + b1_ref[...], 0.0)
    h = (h - m1_ref[...]) * lax.rsqrt(v1_ref[...] + EPS) * g1_ref[...] + be1_ref[...]
    out_ref[...] = jnp.dot(h, w2_ref[...], preferred_element_type=jnp.float32) + b2_ref[...]


def _tc_mlp(xc, xn, bg, bb, bm, bv, w0c, w0n, b0, g0, be0, m0, v0,
            w1, b1, g1, be1, m1, v1, w2, b2):
    TB = 512
    grid = (B // TB,)
    row = lambda i: (i, 0)
    rep = lambda i: (0, 0)
    full = lambda a: pl.BlockSpec(a.shape, rep)
    return pl.pallas_call(
        _mlp_body,
        grid=grid,
        in_specs=[
            pl.BlockSpec((TB, F * D), row),
            pl.BlockSpec((TB, NUM), row),
            full(bg), full(bb), full(bm), full(bv),
            full(w0c), full(w0n), full(b0), full(g0), full(be0), full(m0), full(v0),
            full(w1), full(b1), full(g1), full(be1), full(m1), full(v1),
            full(w2), full(b2),
        ],
        out_specs=pl.BlockSpec((TB, 1), row),
        out_shape=jax.ShapeDtypeStruct((B, 1), jnp.float32),
    )(xc, xn, bg, bb, bm, bv, w0c, w0n, b0, g0, be0, m0, v0,
      w1, b1, g1, be1, m1, v1, w2, b2)


def kernel(x_categorical, x_numerical, emb_tables, bn_num_gamma, bn_num_beta,
           bn_num_mean, bn_num_var, w0, b0, g0, be0, m0, v0,
           w1, b1, g1, be1, m1, v1, w2, b2):
    v = x_categorical.astype(jnp.int32)
    fcol = jnp.arange(F, dtype=jnp.int32)[None, :]
    pk = ((fcol << 20) | (((v >> 7) & 8191) << 7) | (v & 127)).reshape(-1)
    table2 = jnp.swapaxes(emb_tables, 1, 2).reshape(F * D, V)
    g = _tc_gather(table2, pk)
    xc = g.reshape(B, F * D)

    r2 = lambda a: a.reshape(1, -1)
    return _tc_mlp(
        xc, x_numerical,
        r2(bn_num_gamma), r2(bn_num_beta), r2(bn_num_mean), r2(bn_num_var),
        w0[:, :F * D].T, w0[:, F * D:].T, r2(b0), r2(g0), r2(be0), r2(m0), r2(v0),
        w1.T, r2(b1), r2(g1), r2(be1), r2(m1), r2(v1),
        w2.T, r2(b2),
    )


# speedup vs baseline: 11.9889x; 1.0165x over previous
import jax
import jax.numpy as jnp
from jax import lax
from jax.experimental import pallas as pl
from jax.experimental.pallas import tpu as pltpu

B = 4096
F = 26
V = 1000000
D = 16
NUM = 13
H1 = 256
H2 = 128
EPS = 1e-5

R = B * F
LPS = 256
NSTEP = R // LPS


def _gather_body(pk_ref, *refs):
    slab_refs = refs[:LPS]
    out_ref = refs[LPS]
    i = pl.program_id(0)
    eye = (lax.broadcasted_iota(jnp.int32, (128, 128), 0)
           == lax.broadcasted_iota(jnp.int32, (128, 128), 1)).astype(jnp.float32)
    rows = []
    for a in range(LPS // 8):
        sa = jnp.concatenate(
            [s[...] for s in slab_refs[a * 8:(a + 1) * 8]], axis=0)
        blk = lax.dot_general(sa, eye, (((0,), (0,)), ((), ())),
                              preferred_element_type=jnp.float32)
        lcol = jnp.concatenate(
            [jnp.full((D,), pk_ref[i * LPS + a * 8 + j] & 127, jnp.int32)
             for j in range(8)])
        mask = (lax.broadcasted_iota(jnp.int32, (128, 128), 0)
                == lcol.reshape(1, 128)).astype(jnp.float32)
        rows.append(jnp.sum(blk * mask, axis=0))
    out_ref[...] = jnp.stack(rows, axis=0).reshape(1, LPS // 8, 128)


def _tc_gather(table2, pk):
    slab_spec = [
        pl.BlockSpec((D, 128),
                     (lambda j: (lambda i, pk_r:
                                 (pk_r[i * LPS + j] >> 20,
                                  (pk_r[i * LPS + j] >> 7) & 8191)))(j))
        for j in range(LPS)
    ]
    grid_spec = pltpu.PrefetchScalarGridSpec(
        num_scalar_prefetch=1,
        grid=(NSTEP,),
        in_specs=slab_spec,
        out_specs=pl.BlockSpec((1, LPS // 8, 128), lambda i, pk_r: (i, 0, 0)),
    )
    return pl.pallas_call(
        _gather_body,
        grid_spec=grid_spec,
        out_shape=jax.ShapeDtypeStruct((NSTEP, LPS // 8, 128), jnp.float32),
    )(pk, *([table2] * LPS))


def _mlp_body(xc_ref, xn_ref,
              bg_ref, bb_ref, bm_ref, bv_ref,
              w0c_ref, w0n_ref, b0_ref, g0_ref, be0_ref, m0_ref, v0_ref,
              w1_ref, b1_ref, g1_ref, be1_ref, m1_ref, v1_ref,
              w2_ref, b2_ref, out_ref):
    xn = xn_ref[...]
    xnb = (xn - bm_ref[...]) * lax.rsqrt(bv_ref[...] + EPS) * bg_ref[...] + bb_ref[...]
    h = jnp.dot(xc_ref[...], w0c_ref[...], preferred_element_type=jnp.float32)
    h = h + jnp.dot(xnb, w0n_ref[...], preferred_element_type=jnp.float32)
    h = jnp.maximum(h + b0_ref[...], 0.0)
    h = (h - m0_ref[...]) * lax.rsqrt(v0_ref[...] + EPS) * g0_ref[...] + be0_ref[...]
    h = jnp.dot(h, w1_ref[...], preferred_element_type=jnp.float32)
    h = jnp.maximum(h + b1_ref[...], 0.0)
    h = (h - m1_ref[...]) * lax.rsqrt(v1_ref[...] + EPS) * g1_ref[...] + be1_ref[...]
    out_ref[...] = jnp.dot(h, w2_ref[...], preferred_element_type=jnp.float32) + b2_ref[...]


def _tc_mlp(xc, xn, bg, bb, bm, bv, w0c, w0n, b0, g0, be0, m0, v0,
            w1, b1, g1, be1, m1, v1, w2, b2):
    TB = 512
    grid = (B // TB,)
    row = lambda i: (i, 0)
    rep = lambda i: (0, 0)
    full = lambda a: pl.BlockSpec(a.shape, rep)
    return pl.pallas_call(
        _mlp_body,
        grid=grid,
        in_specs=[
            pl.BlockSpec((TB, F * D), row),
            pl.BlockSpec((TB, NUM), row),
            full(bg), full(bb), full(bm), full(bv),
            full(w0c), full(w0n), full(b0), full(g0), full(be0), full(m0), full(v0),
            full(w1), full(b1), full(g1), full(be1), full(m1), full(v1),
            full(w2), full(b2),
        ],
        out_specs=pl.BlockSpec((TB, 1), row),
        out_shape=jax.ShapeDtypeStruct((B, 1), jnp.float32),
    )(xc, xn, bg, bb, bm, bv, w0c, w0n, b0, g0, be0, m0, v0,
      w1, b1, g1, be1, m1, v1, w2, b2)


def kernel(x_categorical, x_numerical, emb_tables, bn_num_gamma, bn_num_beta,
           bn_num_mean, bn_num_var, w0, b0, g0, be0, m0, v0,
           w1, b1, g1, be1, m1, v1, w2, b2):
    v = x_categorical.astype(jnp.int32)
    fcol = jnp.arange(F, dtype=jnp.int32)[None, :]
    pk = ((fcol << 20) | (((v >> 7) & 8191) << 7) | (v & 127)).reshape(-1)
    table2 = jnp.swapaxes(emb_tables, 1, 2).reshape(F * D, V)
    g = _tc_gather(table2, pk)
    xc = g.reshape(B, F * D)

    r2 = lambda a: a.reshape(1, -1)
    return _tc_mlp(
        xc, x_numerical,
        r2(bn_num_gamma), r2(bn_num_beta), r2(bn_num_mean), r2(bn_num_var),
        w0[:, :F * D].T, w0[:, F * D:].T, r2(b0), r2(g0), r2(be0), r2(m0), r2(v0),
        w1.T, r2(b1), r2(g1), r2(be1), r2(m1), r2(v1),
        w2.T, r2(b2),
    )
